# Initial kernel scaffold; baseline (speedup 1.0000x reference)
import jax
import jax.numpy as jnp
from jax.experimental import pallas as pl


def kernel(gdict, flatten_label, llen, cnts, total):
    raise NotImplementedError("write your pallas kernel here")


# speedup vs baseline: 113.1249x; 113.1249x over previous
import functools

import jax
import jax.numpy as jnp
from jax import lax
from jax.experimental import pallas as pl
from jax.experimental.pallas import tpu as pltpu
from jax.experimental.pallas import tpu_sc as plsc

NW = 32
LANES = 16
CHUNK = 8192


def _sc_hist_kernel(llen, n):
    per_w = n // NW
    n_chunks = per_w // CHUNK
    packed = llen // 2
    mesh = plsc.VectorSubcoreMesh(core_axis_name="c", subcore_axis_name="s")

    @functools.partial(
        pl.kernel,
        out_type=jax.ShapeDtypeStruct((NW, llen), jnp.float32),
        mesh=mesh,
        compiler_params=pltpu.CompilerParams(needs_layout_passes=False),
        scratch_types=[
            pltpu.VMEM((packed,), jnp.int32),
            pltpu.VMEM((llen,), jnp.float32),
            pltpu.VMEM((CHUNK,), jnp.int32),
        ],
    )
    def sc_hist(label_hbm, gdp_hbm, out_hbm, gd_v, hist_v, lab_v):
        wid = lax.axis_index("c") * 16 + lax.axis_index("s")
        pltpu.sync_copy(gdp_hbm, gd_v)

        def zbody(i, carry):
            hist_v[pl.ds(i * LANES, LANES)] = jnp.zeros((LANES,), jnp.float32)
            return carry
        lax.fori_loop(0, llen // LANES, zbody, 0)

        ones = jnp.ones((LANES,), jnp.float32)

        def gbody(j, carry):
            lab = lab_v[pl.ds(j * LANES, LANES)]
            word = plsc.load_gather(gd_v, [jnp.right_shift(lab, 1)])
            sel = lax.shift_left((lab & 1), 4)
            mapped = lax.shift_right_logical(word, sel) & 0xFFFF
            plsc.addupdate_scatter(hist_v, [mapped], ones)
            return carry

        for c in range(n_chunks):
            base = wid * per_w + c * CHUNK
            pltpu.sync_copy(label_hbm.at[pl.ds(base, CHUNK)], lab_v)
            lax.fori_loop(0, CHUNK // LANES, gbody, 0)

        pltpu.sync_copy(hist_v, out_hbm.at[wid])

    return sc_hist


def _tc_reduce_kernel(llen, n_f):
    BLK = 8192
    grid = llen // BLK

    def body(total_ref, part_ref, cnts_ref, out_ref):
        tot = total_ref[0, 0] + n_f
        s = jnp.sum(part_ref[...], axis=0) + cnts_ref[...]
        out_ref[...] = jnp.maximum(s / tot, 0.01)

    return pl.pallas_call(
        body,
        grid=(grid,),
        in_specs=[
            pl.BlockSpec(memory_space=pltpu.SMEM),
            pl.BlockSpec((NW, BLK), lambda i: (0, i)),
            pl.BlockSpec((1, BLK), lambda i: (0, i)),
        ],
        out_specs=pl.BlockSpec((1, BLK), lambda i: (0, i)),
        out_shape=jax.ShapeDtypeStruct((1, llen), jnp.float32),
    )


def kernel(gdict, flatten_label, llen, cnts, total):
    llen_static = gdict.shape[0]
    n = flatten_label.shape[0]
    g = gdict.astype(jnp.int32)
    gdp = g[0::2] | lax.shift_left(g[1::2], 16)

    partials = _sc_hist_kernel(llen_static, n)(flatten_label, gdp)

    total2d = jnp.reshape(total.astype(jnp.float32), (1, 1))
    cnts2d = jnp.reshape(cnts[:llen_static], (1, llen_static))
    out = _tc_reduce_kernel(llen_static, float(n))(total2d, partials, cnts2d)
    return jnp.reshape(out, (llen_static,))


# speedup vs baseline: 147.8515x; 1.3070x over previous
import functools

import jax
import jax.numpy as jnp
from jax import lax
from jax.experimental import pallas as pl
from jax.experimental.pallas import tpu as pltpu
from jax.experimental.pallas import tpu_sc as plsc

NW = 32
LANES = 16
CHUNK = 8192


def _sc_hist_kernel(llen, n):
    per_w = n // NW
    n_chunks = per_w // CHUNK
    packed = llen // 2
    mesh = plsc.VectorSubcoreMesh(core_axis_name="c", subcore_axis_name="s")

    @functools.partial(
        pl.kernel,
        out_type=jax.ShapeDtypeStruct((NW, llen), jnp.float32),
        mesh=mesh,
        compiler_params=pltpu.CompilerParams(needs_layout_passes=False),
        scratch_types=[
            pltpu.VMEM((packed,), jnp.int32),
            pltpu.VMEM((llen,), jnp.float32),
            pltpu.VMEM((CHUNK,), jnp.int32),
            pltpu.VMEM((CHUNK,), jnp.int32),
            pltpu.SemaphoreType.DMA,
            pltpu.SemaphoreType.DMA,
            pltpu.SemaphoreType.DMA,
        ],
    )
    def sc_hist(label_hbm, gdp_hbm, out_hbm, gd_v, hist_v, lab0_v, lab1_v,
                gsem, sem0, sem1):
        wid = lax.axis_index("c") * 16 + lax.axis_index("s")
        bufs = (lab0_v, lab1_v)
        sems = (sem0, sem1)

        def start(c):
            base = wid * per_w + c * CHUNK
            return pltpu.async_copy(
                label_hbm.at[pl.ds(base, CHUNK)], bufs[c % 2], sems[c % 2])

        gcopy = pltpu.async_copy(gdp_hbm, gd_v, gsem)
        handles = {0: start(0)}
        if n_chunks > 1:
            handles[1] = start(1)

        ZU = 16
        zero = jnp.zeros((LANES,), jnp.float32)

        def zbody(i, carry):
            base = i * (LANES * ZU)
            for u in range(ZU):
                hist_v[pl.ds(base + u * LANES, LANES)] = zero
            return carry
        lax.fori_loop(0, llen // (LANES * ZU), zbody, 0)

        gcopy.wait()
        ones = jnp.ones((LANES,), jnp.float32)
        U = 8

        for c in range(n_chunks):
            handles[c].wait()
            lab_v = bufs[c % 2]

            def gbody(j, carry):
                gbase = j * (LANES * U)
                for u in range(U):
                    lab = lab_v[pl.ds(gbase + u * LANES, LANES)]
                    word = plsc.load_gather(gd_v, [jnp.right_shift(lab, 1)])
                    sel = lax.shift_left((lab & 1), 4)
                    mapped = lax.shift_right_logical(word, sel) & 0xFFFF
                    plsc.addupdate_scatter(hist_v, [mapped], ones)
                return carry

            lax.fori_loop(0, CHUNK // (LANES * U), gbody, 0)
            if c + 2 < n_chunks:
                handles[c + 2] = start(c + 2)

        pltpu.sync_copy(hist_v, out_hbm.at[wid])

    return sc_hist


def _tc_reduce_kernel(llen, n_f):
    BLK = 8192
    grid = llen // BLK

    def body(total_ref, part_ref, cnts_ref, out_ref):
        tot = total_ref[0, 0] + n_f
        s = jnp.sum(part_ref[...], axis=0) + cnts_ref[...]
        out_ref[...] = jnp.maximum(s / tot, 0.01)

    return pl.pallas_call(
        body,
        grid=(grid,),
        in_specs=[
            pl.BlockSpec(memory_space=pltpu.SMEM),
            pl.BlockSpec((NW, BLK), lambda i: (0, i)),
            pl.BlockSpec((1, BLK), lambda i: (0, i)),
        ],
        out_specs=pl.BlockSpec((1, BLK), lambda i: (0, i)),
        out_shape=jax.ShapeDtypeStruct((1, llen), jnp.float32),
    )


def kernel(gdict, flatten_label, llen, cnts, total):
    llen_static = gdict.shape[0]
    n = flatten_label.shape[0]
    g = gdict.astype(jnp.int32)
    gdp = g[0::2] | lax.shift_left(g[1::2], 16)

    partials = _sc_hist_kernel(llen_static, n)(flatten_label, gdp)

    total2d = jnp.reshape(total.astype(jnp.float32), (1, 1))
    cnts2d = jnp.reshape(cnts[:llen_static], (1, llen_static))
    out = _tc_reduce_kernel(llen_static, float(n))(total2d, partials, cnts2d)
    return jnp.reshape(out, (llen_static,))


# speedup vs baseline: 215.5616x; 1.4580x over previous
import functools

import jax
import jax.numpy as jnp
from jax import lax
from jax.experimental import pallas as pl
from jax.experimental.pallas import tpu as pltpu
from jax.experimental.pallas import tpu_sc as plsc

NW = 32
LANES = 16
CHUNK = 8192


def _sc_hist_kernel(llen, n):
    per_w = n // NW
    n_chunks = per_w // CHUNK
    packed = llen // 2
    mesh = plsc.VectorSubcoreMesh(core_axis_name="c", subcore_axis_name="s")

    @functools.partial(
        pl.kernel,
        out_type=jax.ShapeDtypeStruct((NW, llen), jnp.float32),
        mesh=mesh,
        compiler_params=pltpu.CompilerParams(needs_layout_passes=False),
        scratch_types=[
            pltpu.VMEM((packed,), jnp.int32),
            pltpu.VMEM((llen,), jnp.float32),
            pltpu.VMEM((CHUNK,), jnp.int32),
            pltpu.VMEM((CHUNK,), jnp.int32),
            pltpu.SemaphoreType.DMA,
            pltpu.SemaphoreType.DMA,
            pltpu.SemaphoreType.DMA,
        ],
    )
    def sc_hist(label_hbm, gdp_hbm, out_hbm, gd_v, hist_v, lab0_v, lab1_v,
                gsem, sem0, sem1):
        wid = lax.axis_index("c") * 16 + lax.axis_index("s")
        bufs = (lab0_v, lab1_v)
        sems = (sem0, sem1)

        def start(c):
            base = wid * per_w + c * CHUNK
            return pltpu.async_copy(
                label_hbm.at[pl.ds(base, CHUNK)], bufs[c % 2], sems[c % 2])

        gcopy = pltpu.async_copy(gdp_hbm, gd_v, gsem)
        handles = {0: start(0)}
        if n_chunks > 1:
            handles[1] = start(1)

        zero = jnp.zeros((LANES,), jnp.float32)

        @plsc.parallel_loop(0, llen, LANES, unroll=16)
        def zbody(i):
            hist_v[pl.ds(i, LANES)] = zero

        gcopy.wait()
        ones = jnp.ones((LANES,), jnp.float32)

        for c in range(n_chunks):
            handles[c].wait()
            lab_v = bufs[c % 2]

            @plsc.parallel_loop(0, CHUNK, LANES, unroll=8)
            def gbody(i):
                lab = lab_v[pl.ds(i, LANES)]
                word = plsc.load_gather(gd_v, [jnp.right_shift(lab, 1)])
                sel = lax.shift_left((lab & 1), 4)
                mapped = lax.shift_right_logical(word, sel) & 0xFFFF
                plsc.addupdate_scatter(hist_v, [mapped], ones)

            if c + 2 < n_chunks:
                handles[c + 2] = start(c + 2)

        pltpu.sync_copy(hist_v, out_hbm.at[wid])

    return sc_hist


def _tc_reduce_kernel(llen, n_f):
    BLK = 8192
    grid = llen // BLK

    def body(total_ref, part_ref, cnts_ref, out_ref):
        tot = total_ref[0, 0] + n_f
        s = jnp.sum(part_ref[...], axis=0) + cnts_ref[...]
        out_ref[...] = jnp.maximum(s / tot, 0.01)

    return pl.pallas_call(
        body,
        grid=(grid,),
        in_specs=[
            pl.BlockSpec(memory_space=pltpu.SMEM),
            pl.BlockSpec((NW, BLK), lambda i: (0, i)),
            pl.BlockSpec((1, BLK), lambda i: (0, i)),
        ],
        out_specs=pl.BlockSpec((1, BLK), lambda i: (0, i)),
        out_shape=jax.ShapeDtypeStruct((1, llen), jnp.float32),
    )


def kernel(gdict, flatten_label, llen, cnts, total):
    llen_static = gdict.shape[0]
    n = flatten_label.shape[0]
    g = gdict.astype(jnp.int32)
    gdp = g[0::2] | lax.shift_left(g[1::2], 16)

    partials = _sc_hist_kernel(llen_static, n)(flatten_label, gdp)

    total2d = jnp.reshape(total.astype(jnp.float32), (1, 1))
    cnts2d = jnp.reshape(cnts[:llen_static], (1, llen_static))
    out = _tc_reduce_kernel(llen_static, float(n))(total2d, partials, cnts2d)
    return jnp.reshape(out, (llen_static,))


# speedup vs baseline: 276.0766x; 1.2807x over previous
import functools

import jax
import jax.numpy as jnp
from jax import lax
from jax.experimental import pallas as pl
from jax.experimental.pallas import tpu as pltpu
from jax.experimental.pallas import tpu_sc as plsc

NW = 32
LANES = 16
CHUNK = 8192


def _sc_hist_kernel(llen, n):
    per_w = n // NW
    n_chunks = per_w // CHUNK
    half = llen // 2
    mesh = plsc.VectorSubcoreMesh(core_axis_name="c", subcore_axis_name="s")

    @functools.partial(
        pl.kernel,
        out_type=jax.ShapeDtypeStruct((NW, half), jnp.int32),
        mesh=mesh,
        compiler_params=pltpu.CompilerParams(needs_layout_passes=False),
        scratch_types=[
            pltpu.VMEM((llen,), jnp.int32),
            pltpu.VMEM((half,), jnp.int32),
            pltpu.VMEM((CHUNK,), jnp.int32),
            pltpu.VMEM((CHUNK,), jnp.int32),
            pltpu.SemaphoreType.DMA,
            pltpu.SemaphoreType.DMA,
            pltpu.SemaphoreType.DMA,
        ],
    )
    def sc_hist(label_hbm, gd_hbm, out_hbm, gd_v, hist_v, lab0_v, lab1_v,
                gsem, sem0, sem1):
        wid = lax.axis_index("c") * 16 + lax.axis_index("s")
        bufs = (lab0_v, lab1_v)
        sems = (sem0, sem1)

        def start(c):
            base = wid * per_w + c * CHUNK
            return pltpu.async_copy(
                label_hbm.at[pl.ds(base, CHUNK)], bufs[c % 2], sems[c % 2])

        gcopy = pltpu.async_copy(gd_hbm, gd_v, gsem)
        handles = {0: start(0)}
        if n_chunks > 1:
            handles[1] = start(1)

        zero = jnp.zeros((LANES,), jnp.int32)

        @plsc.parallel_loop(0, half, LANES, unroll=16)
        def zbody(i):
            hist_v[pl.ds(i, LANES)] = zero

        gcopy.wait()
        one = jnp.full((LANES,), 1, jnp.int32)

        for c in range(n_chunks):
            handles[c].wait()
            lab_v = bufs[c % 2]

            @plsc.parallel_loop(0, CHUNK, LANES, unroll=8)
            def gbody(i):
                lab = lab_v[pl.ds(i, LANES)]
                mapped = plsc.load_gather(gd_v, [lab])
                word = mapped & 0x7FFF
                inc = lax.shift_left(
                    one, lax.shift_left(lax.shift_right_logical(mapped, 15), 4))
                plsc.addupdate_scatter(hist_v, [word], inc)

            if c + 2 < n_chunks:
                handles[c + 2] = start(c + 2)

        pltpu.sync_copy(hist_v, out_hbm.at[wid])

    return sc_hist


def _tc_reduce_kernel(llen, n_f):
    BLK = 8192
    half = llen // 2
    grid = half // BLK

    def body(total_ref, part_ref, cnts_ref, out_ref):
        tot = total_ref[0, 0] + n_f
        p = part_ref[...]
        s_lo = jnp.sum(p & 0xFFFF, axis=0).astype(jnp.float32)
        s_hi = jnp.sum(lax.shift_right_logical(p, 16), axis=0).astype(jnp.float32)
        out_ref[0, :] = jnp.maximum((s_lo + cnts_ref[0, :]) / tot, 0.01)
        out_ref[1, :] = jnp.maximum((s_hi + cnts_ref[1, :]) / tot, 0.01)

    return pl.pallas_call(
        body,
        grid=(grid,),
        in_specs=[
            pl.BlockSpec(memory_space=pltpu.SMEM),
            pl.BlockSpec((NW, BLK), lambda i: (0, i)),
            pl.BlockSpec((2, BLK), lambda i: (0, i)),
        ],
        out_specs=pl.BlockSpec((2, BLK), lambda i: (0, i)),
        out_shape=jax.ShapeDtypeStruct((2, half), jnp.float32),
    )


def kernel(gdict, flatten_label, llen, cnts, total):
    llen_static = gdict.shape[0]
    n = flatten_label.shape[0]

    partials = _sc_hist_kernel(llen_static, n)(
        flatten_label, gdict.astype(jnp.int32))

    total2d = jnp.reshape(total.astype(jnp.float32), (1, 1))
    cnts2d = jnp.reshape(cnts[:llen_static], (2, llen_static // 2))
    out = _tc_reduce_kernel(llen_static, float(n))(total2d, partials, cnts2d)
    return jnp.reshape(out, (llen_static,))
